# baseline (speedup 1.0000x reference)
import functools

import jax
import jax.numpy as jnp
from jax import lax
from jax.experimental import pallas as pl
from jax.experimental.pallas import tpu as pltpu
from jax.experimental.pallas import tpu_sc as plsc

N0, N1, N2, N3, N4 = 32768, 8192, 2048, 512, 128
C = 256
XP = 16
NC, NS = 2, 16
NW = NC * NS
L = 16

_SC_MESH = dict(mesh=plsc.VectorSubcoreMesh(core_axis_name="c", subcore_axis_name="s"))


def _wid():
    return lax.axis_index("c") * NS + lax.axis_index("s")


@functools.partial(
    pl.kernel,
    out_type=(
        jax.ShapeDtypeStruct((N1, XP), jnp.float32),
        jax.ShapeDtypeStruct((N2, XP), jnp.float32),
        jax.ShapeDtypeStruct((N3, XP), jnp.float32),
        jax.ShapeDtypeStruct((N4, XP), jnp.float32),
    ),
    scratch_types=[
        pltpu.VMEM((N1 // NW,), jnp.int32),
        pltpu.VMEM((N1 // NW,), jnp.int32),
        pltpu.VMEM((N1 // NW + 1, 128), jnp.float32),
        pltpu.VMEM((N1 // NW, XP), jnp.float32),
        pltpu.SemaphoreType.DMA,
    ],
    **_SC_MESH,
)
def _sc_gather_xyz(x04r, i11, i9, i7, i5, o1, o2, o3, o4, idx_v, row_v, rows_v,
                   out_v, sem):
    w = _wid()
    iota16 = lax.broadcasted_iota(jnp.int32, (L,), 0)
    cmask = iota16 < 3

    def gather(idx_hbm, out_hbm, per, base):
        base = pl.multiple_of(base, 64)
        pltpu.sync_copy(idx_hbm.at[pl.ds(base, per)], idx_v.at[pl.ds(0, per)])

        def rowchunk(cc, _):
            v = idx_v[pl.ds(cc * L, L)]
            row_v[pl.ds(cc * L, L)] = lax.shift_right_logical(v, 5)
            return 0

        lax.fori_loop(0, per // L, rowchunk, 0)
        pltpu.async_copy(
            x04r.at[row_v.at[pl.ds(0, per)]], rows_v.at[pl.ds(0, per)], sem
        ).wait()

        def extchunk(cc, _):
            iv = idx_v[pl.ds(cc * L, L)]
            col = (iv & 31) * 4
            for l in range(L):
                j = cc * L + l
                v = rows_v[j, pl.ds(col[l], L)]
                out_v[j, pl.ds(0, XP)] = jnp.where(cmask, v, 0.0)
            return 0

        lax.fori_loop(0, per // L, extchunk, 0)
        pltpu.sync_copy(out_v.at[pl.ds(0, per)], out_hbm.at[pl.ds(base, per)])

    gather(i11, o1, N1 // NW, w * (N1 // NW))
    gather(i9, o2, N2 // NW, w * (N2 // NW))
    @pl.when(w < N3 // 64)
    def _():
        gather(i7, o3, 64, w * 64)

    @pl.when(w < N4 // 64)
    def _():
        gather(i5, o4, 64, w * 64)


def _argmin_rows(q, rT):
    dot = jnp.dot(q, rT, preferred_element_type=jnp.float32)
    qn = jnp.sum(q * q, axis=1, keepdims=True)
    rn = jnp.sum(rT * rT, axis=0, keepdims=True)
    e = qn + rn - 2.0 * dot
    minv = jnp.min(e, axis=1, keepdims=True)
    ii = lax.broadcasted_iota(jnp.int32, e.shape, 1)
    return jnp.min(jnp.where(e <= minv, ii, jnp.int32(2**30)), axis=1)


_KNN_BLK = 512


def _tc_knn12_body(q_ref, r_ref, o_ref, rT_ref):
    @pl.when(pl.program_id(0) == 0)
    def _():
        rT_ref[...] = r_ref[...].T

    o_ref[0, 0, :] = _argmin_rows(q_ref[...], rT_ref[...])


def _tc_knn12(x1p, x2p):
    nblk = N1 // _KNN_BLK
    out = pl.pallas_call(
        _tc_knn12_body,
        grid=(nblk,),
        in_specs=[
            pl.BlockSpec((_KNN_BLK, XP), lambda i: (i, 0)),
            pl.BlockSpec((N2, XP), lambda i: (0, 0)),
        ],
        out_specs=pl.BlockSpec((1, 1, _KNN_BLK), lambda i: (i, 0, 0)),
        out_shape=jax.ShapeDtypeStruct((nblk, 1, _KNN_BLK), jnp.int32),
        scratch_shapes=[pltpu.VMEM((XP, N2), jnp.float32)],
    )(x1p, x2p)
    return out.reshape(N1)


def _tc_levels_body(
    x0_ref, x1pk_ref, x2pk_ref, x3pk_ref, x4pk_ref,
    wall_ref, wm4_ref, wm3_ref, wm2_ref, wm1_ref, wm0_ref,
    wp4_ref, wp3_ref, wp2_ref, wp1_ref, wp0_ref, bias_ref,
    f4o_ref, f3o_ref, f2o_ref, t2o_ref, qro_ref, s0o_ref,
):
    x0 = x0_ref[...]
    x1p = x1pk_ref[...]
    x2p = x2pk_ref[...]
    x3p = x3pk_ref[...]
    x4p = x4pk_ref[...]
    wall = wall_ref[...]
    b_all = bias_ref[0:1, :]

    def dot(a, b):
        return jnp.dot(a, b, preferred_element_type=jnp.float32)

    def padw(m):
        return jnp.concatenate([m, jnp.zeros((XP - 3, C), jnp.float32)], axis=0)

    wp4a, wp4b = wp4_ref[:C, :], wp4_ref[C:, :]
    f4 = dot(x0[:N4], wall) + b_all
    m4 = jnp.max(f4, axis=0, keepdims=True)
    M4 = dot(wm4_ref[...], wp4a)
    A4 = dot(wall, wp4b)
    c4 = dot(bias_ref[1:2, :], wp4a) + dot(b_all, wp4b) + bias_ref[5:6, :]
    feat4 = dot(m4, wp4a) + dot(x4p, padw(M4)) + dot(x0[:N4], A4) + c4
    f4o_ref[...] = feat4

    wp3a, wp3b = wp3_ref[:C, :], wp3_ref[C:, :]
    M3 = dot(wm3_ref[...], wp3a)
    A3 = dot(wall, wp3b)
    c3 = dot(bias_ref[2:3, :], wp3a) + dot(b_all, wp3b) + bias_ref[6:7, :]
    G4 = dot(feat4, wp3a) - dot(x4p, padw(M3)) + c3
    k34 = _argmin_rows(x3p, x4p.T)
    oh34 = (k34[:, None] == lax.broadcasted_iota(jnp.int32, (N3, N4), 1)).astype(
        jnp.float32
    )
    feat3 = dot(oh34, G4) + dot(x3p, padw(M3)) + dot(x0[:N3], A3)
    f3o_ref[...] = feat3

    wp2a, wp2b = wp2_ref[:C, :], wp2_ref[C:, :]
    M2 = dot(wm2_ref[...], wp2a)
    A2 = dot(wall, wp2b)
    c2 = dot(bias_ref[3:4, :], wp2a) + dot(b_all, wp2b) + bias_ref[7:8, :]
    G3 = dot(feat3, wp2a) - dot(x3p, padw(M2)) + c2
    k23 = _argmin_rows(x2p, x3p.T)
    oh23 = (k23[:, None] == lax.broadcasted_iota(jnp.int32, (N2, N3), 1)).astype(
        jnp.float32
    )
    feat2 = dot(oh23, G3) + dot(x2p, padw(M2)) + dot(x0[:N2], A2)
    f2o_ref[...] = feat2

    wp1a, wp1b = wp1_ref[:C, :], wp1_ref[C:, :]
    wp0a, wp0b = wp0_ref[:C, :], wp0_ref[C:, :]
    M1 = dot(wm1_ref[...], wp1a)
    A1 = dot(wall, wp1b)
    c1 = dot(bias_ref[4:5, :], wp1a) + dot(b_all, wp1b) + bias_ref[8:9, :]
    G2 = dot(feat2, wp1a) - dot(x2p, padw(M1)) + c1
    t2o_ref[:, :C] = G2
    t2o_ref[:, C:] = dot(G2, wp0a)
    Q1 = dot(x1p, padw(M1)) + dot(x0, A1)
    qro_ref[:, :C] = Q1
    M0 = dot(wm0_ref[...], wp0a)
    A0 = dot(wall, wp0b)
    c0 = (dot(bias_ref[9:10, :], wp0a) + dot(b_all, wp0b) + bias_ref[10:11, :])
    qro_ref[:, C:] = dot(Q1, wp0a) - dot(x1p, padw(M0)) + c0
    s0o_ref[...] = jnp.concatenate(
        [M0 + A0, jnp.zeros((8 - 3, C), jnp.float32)], axis=0
    ).reshape(2, 4 * C)


def _tc_levels(x0, x1pk, x2pk, x3pk, x4pk, wall, wm4, wm3, wm2, wm1, wm0,
               wp4, wp3, wp2, wp1, wp0, bias):
    args = (x0, x1pk, x2pk, x3pk, x4pk, wall, wm4, wm3, wm2, wm1, wm0,
            wp4, wp3, wp2, wp1, wp0, bias)
    specs = [pl.BlockSpec((N1, 3), lambda i: (0, 0))]
    for a in args[1:]:
        specs.append(pl.BlockSpec(a.shape, lambda i, nd=a.ndim: (0,) * nd))
    out_shapes = (
            jax.ShapeDtypeStruct((N4, C), jnp.float32),
            jax.ShapeDtypeStruct((N3, C), jnp.float32),
            jax.ShapeDtypeStruct((N2, C), jnp.float32),
            jax.ShapeDtypeStruct((N2, 2 * C), jnp.float32),
            jax.ShapeDtypeStruct((N1, 2 * C), jnp.float32),
            jax.ShapeDtypeStruct((2, 4 * C), jnp.float32),
        )
    return pl.pallas_call(
        _tc_levels_body,
        grid=(1,),
        in_specs=specs,
        out_specs=[pl.BlockSpec(o.shape, lambda i, nd=o.ndim: (0,) * nd)
                   for o in out_shapes],
        out_shape=out_shapes,
    )(*args)


_F1_SUB = 16
_F1_NSUB = N1 // NW // _F1_SUB
_NB = 3


def _sc_pipeline(n_sub, start, compute, writeback):
    pend = {}
    outp = {}
    pend[0] = start(0, 0)
    if n_sub > 1:
        pend[1] = start(1, 1)
    for s in range(n_sub):
        b = s % _NB
        if s + 2 < n_sub:
            nb = (s + 2) % _NB
            if nb in outp:
                for cp in outp.pop(nb):
                    cp.wait()
            pend[nb] = start(s + 2, nb)
        for cp in pend.pop(b):
            cp.wait()
        if b in outp:
            for cp in outp.pop(b):
                cp.wait()
        compute(s, b)
        outp[b] = writeback(s, b)
    for cps in outp.values():
        for cp in cps:
            cp.wait()


@functools.partial(
    pl.kernel,
    out_type=(
        jax.ShapeDtypeStruct((N1, C), jnp.float32),
        jax.ShapeDtypeStruct((N1, C), jnp.float32),
    ),
    scratch_types=[
        pltpu.VMEM((N1 // NW,), jnp.int32),
        pltpu.VMEM((_NB, _F1_SUB, 2 * C), jnp.float32),
        pltpu.VMEM((_NB, _F1_SUB, 2 * C), jnp.float32),
        pltpu.VMEM((_NB, _F1_SUB, C), jnp.float32),
        pltpu.VMEM((_NB, _F1_SUB, C), jnp.float32),
        pltpu.SemaphoreType.DMA,
        pltpu.SemaphoreType.DMA,
        pltpu.SemaphoreType.DMA,
        pltpu.SemaphoreType.DMA,
        pltpu.SemaphoreType.DMA,
        pltpu.SemaphoreType.DMA,
    ],
    **_SC_MESH,
)
def _sc_feat1g1(t2, k12, qr, f1o, g1o, idx_v, g_v, q_v, fo_v, go_v,
                sg0, sg1, sg2, sq0, sq1, sq2):
    w = _wid()
    sg, sq = (sg0, sg1, sg2), (sq0, sq1, sq2)
    base0 = pl.multiple_of(w * (N1 // NW), N1 // NW)
    pltpu.sync_copy(k12.at[pl.ds(base0, N1 // NW)], idx_v)

    def start(s, b):
        base = pl.multiple_of(base0 + s * _F1_SUB, _F1_SUB)
        return (
            pltpu.async_copy(
                t2.at[idx_v.at[pl.ds(s * _F1_SUB, _F1_SUB)]], g_v.at[b], sg[b]
            ),
            pltpu.async_copy(qr.at[pl.ds(base, _F1_SUB)], q_v.at[b], sq[b]),
        )

    def compute(s, b):
        def row(r, _):
            for c in range(C // L):
                sl = pl.ds(c * L, L)
                sh = pl.ds(C + c * L, L)
                fo_v[b, r, sl] = g_v[b, r, sl] + q_v[b, r, sl]
                go_v[b, r, sl] = g_v[b, r, sh] + q_v[b, r, sh]
            return 0

        lax.fori_loop(0, _F1_SUB, row, 0)

    def writeback(s, b):
        base = pl.multiple_of(base0 + s * _F1_SUB, _F1_SUB)
        return (
            pltpu.async_copy(fo_v.at[b], f1o.at[pl.ds(base, _F1_SUB)], sg[b]),
            pltpu.async_copy(go_v.at[b], g1o.at[pl.ds(base, _F1_SUB)], sq[b]),
        )

    _sc_pipeline(_F1_NSUB, start, compute, writeback)


_F0_SUB = 64
_F0_NSUB = N0 // NW // _F0_SUB


@functools.partial(
    pl.kernel,
    out_type=jax.ShapeDtypeStruct((N0, C), jnp.float32),
    scratch_types=[
        pltpu.VMEM((N0 // NW,), jnp.int32),
        pltpu.VMEM((_NB, _F0_SUB, C), jnp.float32),
        pltpu.VMEM((_NB, _F0_SUB, C), jnp.float32),
        pltpu.VMEM((N0 // NW // 32 + 1, 128), jnp.float32),
        pltpu.VMEM((2, 4 * C), jnp.float32),
        pltpu.SemaphoreType.DMA,
        pltpu.SemaphoreType.DMA,
        pltpu.SemaphoreType.DMA,
    ],
    **_SC_MESH,
)
def _sc_feat0(g1, idx0, x04r, s0p, out, idx_v, g_v, o_v, x_v, s_v,
              sg0, sg1, sg2):
    w = _wid()
    sg = (sg0, sg1, sg2)
    base0 = pl.multiple_of(w * (N0 // NW), N0 // NW)
    pltpu.sync_copy(idx0.at[pl.ds(base0, N0 // NW)], idx_v)
    pltpu.sync_copy(s0p, s_v)
    pltpu.sync_copy(
        x04r.at[pl.ds(pl.multiple_of(base0 // 32, 8), N0 // NW // 32)],
        x_v.at[pl.ds(0, N0 // NW // 32)],
    )
    s0 = [s_v[0, pl.ds(c * L, L)] for c in range(C // L)]
    s1 = [s_v[0, pl.ds(C + c * L, L)] for c in range(C // L)]
    s2 = [s_v[0, pl.ds(2 * C + c * L, L)] for c in range(C // L)]

    def start(s, b):
        return (
            pltpu.async_copy(
                g1.at[idx_v.at[pl.ds(s * _F0_SUB, _F0_SUB)]], g_v.at[b], sg[b]
            ),
        )

    def compute(s, b):
        def row(r, _):
            g = s * _F0_SUB + r
            xrow = x_v[g // 32, pl.ds((g % 32) * 4, L)]
            x, y, z = xrow[0], xrow[1], xrow[2]
            for c in range(C // L):
                sl = pl.ds(c * L, L)
                o_v[b, r, sl] = g_v[b, r, sl] + x * s0[c] + y * s1[c] + z * s2[c]
            return 0

        lax.fori_loop(0, _F0_SUB, row, 0)

    def writeback(s, b):
        base = pl.multiple_of(base0 + s * _F0_SUB, _F0_SUB)
        return (
            pltpu.async_copy(o_v.at[b], out.at[pl.ds(base, _F0_SUB)], sg[b]),
        )

    _sc_pipeline(_F0_NSUB, start, compute, writeback)


def kernel(xyz0, idx0, idx5, idx7, idx9, idx11, W_all, b_all, W_m4, b_m4,
           W_m3, b_m3, W_m2, b_m2, W_m1, b_m1, W_m0, b_m0, W_p4, b_p4,
           W_p3, b_p3, W_p2, b_p2, W_p1, b_p1, W_p0, b_p0):
    f32 = jnp.float32
    xyz0 = xyz0.astype(f32)
    x04r = jnp.pad(xyz0, ((0, 0), (0, 1))).reshape(N0 // 32, 128)

    i0 = idx0.astype(jnp.int32)
    i5, i7, i9, i11 = (i.astype(jnp.int32) for i in (idx5, idx7, idx9, idx11))

    x1pk, x2pk, x3pk, x4pk = _sc_gather_xyz(x04r, i11, i9, i7, i5)

    k12 = _tc_knn12(x1pk, x2pk)

    bias = jnp.stack([b_all, b_m4, b_m3, b_m2, b_m1, b_p4, b_p3, b_p2, b_p1,
                      b_m0, b_p0])
    feat4, feat3, feat2, T2, QR, S0 = _tc_levels(
        xyz0, x1pk, x2pk, x3pk, x4pk,
        W_all.astype(f32), W_m4.astype(f32), W_m3.astype(f32),
        W_m2.astype(f32), W_m1.astype(f32), W_m0.astype(f32),
        W_p4, W_p3, W_p2, W_p1, W_p0, bias)

    feat1, G1 = _sc_feat1g1(T2, k12, QR)

    feat0 = _sc_feat0(G1, i0, x04r, S0)

    return (feat4, feat3, feat2, feat1, feat0)


# speedup vs baseline: 1.0669x; 1.0669x over previous
import functools

import jax
import jax.numpy as jnp
from jax import lax
from jax.experimental import pallas as pl
from jax.experimental.pallas import tpu as pltpu
from jax.experimental.pallas import tpu_sc as plsc

N0, N1, N2, N3, N4 = 32768, 8192, 2048, 512, 128
C = 256
XP = 16
NC, NS = 2, 16
NW = NC * NS
L = 16

_SC_MESH = dict(mesh=plsc.VectorSubcoreMesh(core_axis_name="c", subcore_axis_name="s"))


def _wid():
    return lax.axis_index("c") * NS + lax.axis_index("s")


@functools.partial(
    pl.kernel,
    out_type=(
        jax.ShapeDtypeStruct((N1, XP), jnp.float32),
        jax.ShapeDtypeStruct((N2, XP), jnp.float32),
        jax.ShapeDtypeStruct((N3, XP), jnp.float32),
        jax.ShapeDtypeStruct((N4, XP), jnp.float32),
    ),
    scratch_types=[
        pltpu.VMEM((N1 // NW,), jnp.int32),
        pltpu.VMEM((N1 // NW,), jnp.int32),
        pltpu.VMEM((N1 // NW + 1, 128), jnp.float32),
        pltpu.VMEM((N1 // NW, XP), jnp.float32),
        pltpu.SemaphoreType.DMA,
    ],
    **_SC_MESH,
)
def _sc_gather_xyz(x04r, i11, i9, i7, i5, o1, o2, o3, o4, idx_v, row_v, rows_v,
                   out_v, sem):
    w = _wid()
    iota16 = lax.broadcasted_iota(jnp.int32, (L,), 0)
    cmask = iota16 < 3

    def gather(idx_hbm, out_hbm, per, base):
        base = pl.multiple_of(base, 64)
        pltpu.sync_copy(idx_hbm.at[pl.ds(base, per)], idx_v.at[pl.ds(0, per)])

        def rowchunk(cc, _):
            v = idx_v[pl.ds(cc * L, L)]
            row_v[pl.ds(cc * L, L)] = lax.shift_right_logical(v, 5)
            return 0

        lax.fori_loop(0, per // L, rowchunk, 0)
        pltpu.async_copy(
            x04r.at[row_v.at[pl.ds(0, per)]], rows_v.at[pl.ds(0, per)], sem
        ).wait()

        def extchunk(cc, _):
            iv = idx_v[pl.ds(cc * L, L)]
            col = (iv & 31) * 4
            for l in range(L):
                j = cc * L + l
                v = rows_v[j, pl.ds(col[l], L)]
                out_v[j, pl.ds(0, XP)] = jnp.where(cmask, v, 0.0)
            return 0

        lax.fori_loop(0, per // L, extchunk, 0)
        pltpu.sync_copy(out_v.at[pl.ds(0, per)], out_hbm.at[pl.ds(base, per)])

    gather(i11, o1, N1 // NW, w * (N1 // NW))
    gather(i9, o2, N2 // NW, w * (N2 // NW))
    @pl.when(w < N3 // 64)
    def _():
        gather(i7, o3, 64, w * 64)

    @pl.when(w < N4 // 64)
    def _():
        gather(i5, o4, 64, w * 64)


def _argmin_rows(q, rT):
    dot = jnp.dot(q, rT, preferred_element_type=jnp.float32)
    qn = jnp.sum(q * q, axis=1, keepdims=True)
    rn = jnp.sum(rT * rT, axis=0, keepdims=True)
    e = qn + rn - 2.0 * dot
    minv = jnp.min(e, axis=1, keepdims=True)
    ii = lax.broadcasted_iota(jnp.int32, e.shape, 1)
    return jnp.min(jnp.where(e <= minv, ii, jnp.int32(2**30)), axis=1)


_KNN_BLK = 512


def _tc_knn12_body(q_ref, r_ref, o_ref, rT_ref):
    @pl.when(pl.program_id(0) == 0)
    def _():
        rT_ref[...] = r_ref[...].T

    o_ref[0, 0, :] = _argmin_rows(q_ref[...], rT_ref[...])


def _tc_knn12(x1p, x2p):
    nblk = N1 // _KNN_BLK
    out = pl.pallas_call(
        _tc_knn12_body,
        grid=(nblk,),
        in_specs=[
            pl.BlockSpec((_KNN_BLK, XP), lambda i: (i, 0)),
            pl.BlockSpec((N2, XP), lambda i: (0, 0)),
        ],
        out_specs=pl.BlockSpec((1, 1, _KNN_BLK), lambda i: (i, 0, 0)),
        out_shape=jax.ShapeDtypeStruct((nblk, 1, _KNN_BLK), jnp.int32),
        scratch_shapes=[pltpu.VMEM((XP, N2), jnp.float32)],
    )(x1p, x2p)
    return out.reshape(N1)


def _tc_levels_body(
    x0_ref, x1pk_ref, x2pk_ref, x3pk_ref, x4pk_ref,
    wall_ref, wm4_ref, wm3_ref, wm2_ref, wm1_ref, wm0_ref,
    wp4_ref, wp3_ref, wp2_ref, wp1_ref, wp0_ref, bias_ref,
    f4o_ref, f3o_ref, f2o_ref, t2o_ref, qro_ref, s0o_ref,
):
    x0 = x0_ref[...]
    x1p = x1pk_ref[...]
    x2p = x2pk_ref[...]
    x3p = x3pk_ref[...]
    x4p = x4pk_ref[...]
    wall = wall_ref[...]
    b_all = bias_ref[0:1, :]

    def dot(a, b):
        return jnp.dot(a, b, preferred_element_type=jnp.float32)

    def padw(m):
        return jnp.concatenate([m, jnp.zeros((XP - 3, C), jnp.float32)], axis=0)

    wp4a, wp4b = wp4_ref[:C, :], wp4_ref[C:, :]
    f4 = dot(x0[:N4], wall) + b_all
    m4 = jnp.max(f4, axis=0, keepdims=True)
    M4 = dot(wm4_ref[...], wp4a)
    A4 = dot(wall, wp4b)
    c4 = dot(bias_ref[1:2, :], wp4a) + dot(b_all, wp4b) + bias_ref[5:6, :]
    feat4 = dot(m4, wp4a) + dot(x4p, padw(M4)) + dot(x0[:N4], A4) + c4
    f4o_ref[...] = feat4

    wp3a, wp3b = wp3_ref[:C, :], wp3_ref[C:, :]
    M3 = dot(wm3_ref[...], wp3a)
    A3 = dot(wall, wp3b)
    c3 = dot(bias_ref[2:3, :], wp3a) + dot(b_all, wp3b) + bias_ref[6:7, :]
    G4 = dot(feat4, wp3a) - dot(x4p, padw(M3)) + c3
    k34 = _argmin_rows(x3p, x4p.T)
    oh34 = (k34[:, None] == lax.broadcasted_iota(jnp.int32, (N3, N4), 1)).astype(
        jnp.float32
    )
    feat3 = dot(oh34, G4) + dot(x3p, padw(M3)) + dot(x0[:N3], A3)
    f3o_ref[...] = feat3

    wp2a, wp2b = wp2_ref[:C, :], wp2_ref[C:, :]
    M2 = dot(wm2_ref[...], wp2a)
    A2 = dot(wall, wp2b)
    c2 = dot(bias_ref[3:4, :], wp2a) + dot(b_all, wp2b) + bias_ref[7:8, :]
    G3 = dot(feat3, wp2a) - dot(x3p, padw(M2)) + c2
    k23 = _argmin_rows(x2p, x3p.T)
    oh23 = (k23[:, None] == lax.broadcasted_iota(jnp.int32, (N2, N3), 1)).astype(
        jnp.float32
    )
    feat2 = dot(oh23, G3) + dot(x2p, padw(M2)) + dot(x0[:N2], A2)
    f2o_ref[...] = feat2

    wp1a, wp1b = wp1_ref[:C, :], wp1_ref[C:, :]
    wp0a, wp0b = wp0_ref[:C, :], wp0_ref[C:, :]
    M1 = dot(wm1_ref[...], wp1a)
    A1 = dot(wall, wp1b)
    c1 = dot(bias_ref[4:5, :], wp1a) + dot(b_all, wp1b) + bias_ref[8:9, :]
    G2 = dot(feat2, wp1a) - dot(x2p, padw(M1)) + c1
    t2o_ref[:, :C] = G2
    t2o_ref[:, C:] = dot(G2, wp0a)
    Q1 = dot(x1p, padw(M1)) + dot(x0, A1)
    qro_ref[:, :C] = Q1
    M0 = dot(wm0_ref[...], wp0a)
    A0 = dot(wall, wp0b)
    c0 = (dot(bias_ref[9:10, :], wp0a) + dot(b_all, wp0b) + bias_ref[10:11, :])
    qro_ref[:, C:] = dot(Q1, wp0a) - dot(x1p, padw(M0)) + c0
    s0o_ref[...] = jnp.concatenate(
        [M0 + A0, jnp.zeros((8 - 3, C), jnp.float32)], axis=0
    ).reshape(2, 4 * C)


def _tc_levels(x0, x1pk, x2pk, x3pk, x4pk, wall, wm4, wm3, wm2, wm1, wm0,
               wp4, wp3, wp2, wp1, wp0, bias):
    args = (x0, x1pk, x2pk, x3pk, x4pk, wall, wm4, wm3, wm2, wm1, wm0,
            wp4, wp3, wp2, wp1, wp0, bias)
    specs = [pl.BlockSpec((N1, 3), lambda i: (0, 0))]
    for a in args[1:]:
        specs.append(pl.BlockSpec(a.shape, lambda i, nd=a.ndim: (0,) * nd))
    out_shapes = (
            jax.ShapeDtypeStruct((N4, C), jnp.float32),
            jax.ShapeDtypeStruct((N3, C), jnp.float32),
            jax.ShapeDtypeStruct((N2, C), jnp.float32),
            jax.ShapeDtypeStruct((N2, 2 * C), jnp.float32),
            jax.ShapeDtypeStruct((N1, 2 * C), jnp.float32),
            jax.ShapeDtypeStruct((2, 4 * C), jnp.float32),
        )
    return pl.pallas_call(
        _tc_levels_body,
        grid=(1,),
        in_specs=specs,
        out_specs=[pl.BlockSpec(o.shape, lambda i, nd=o.ndim: (0,) * nd)
                   for o in out_shapes],
        out_shape=out_shapes,
    )(*args)


_F1_SUB = 32
_F1_NSUB = N1 // NW // _F1_SUB


@functools.partial(
    pl.kernel,
    out_type=(
        jax.ShapeDtypeStruct((N1, C), jnp.float32),
        jax.ShapeDtypeStruct((N1, C), jnp.float32),
    ),
    scratch_types=[
        pltpu.VMEM((N1 // NW,), jnp.int32),
        pltpu.VMEM((2, _F1_SUB, 2 * C), jnp.float32),
        pltpu.VMEM((2, _F1_SUB, 2 * C), jnp.float32),
        pltpu.VMEM((2, _F1_SUB, C), jnp.float32),
        pltpu.VMEM((2, _F1_SUB, C), jnp.float32),
        pltpu.SemaphoreType.DMA,
        pltpu.SemaphoreType.DMA,
        pltpu.SemaphoreType.DMA,
        pltpu.SemaphoreType.DMA,
        pltpu.SemaphoreType.DMA,
        pltpu.SemaphoreType.DMA,
    ],
    **_SC_MESH,
)
def _sc_feat1g1(t2, k12, qr, f1o, g1o, idx_v, g_v, q_v, fo_v, go_v,
                sg0, sg1, sq0, sq1, so0, so1):
    w = _wid()
    sg, sq, so = (sg0, sg1), (sq0, sq1), (so0, so1)
    base0 = pl.multiple_of(w * (N1 // NW), N1 // NW)
    pltpu.sync_copy(k12.at[pl.ds(base0, N1 // NW)], idx_v)

    def start(s, b):
        base = pl.multiple_of(base0 + s * _F1_SUB, _F1_SUB)
        return (
            pltpu.async_copy(
                t2.at[idx_v.at[pl.ds(s * _F1_SUB, _F1_SUB)]], g_v.at[b], sg[b]
            ),
            pltpu.async_copy(qr.at[pl.ds(base, _F1_SUB)], q_v.at[b], sq[b]),
        )

    pend = [None, None]
    outp = [None, None]
    pend[0] = start(0, 0)
    pend[1] = start(1, 1)
    for s in range(_F1_NSUB):
        b = s % 2
        base = pl.multiple_of(base0 + s * _F1_SUB, _F1_SUB)
        for cp in pend[b]:
            cp.wait()
        if outp[b] is not None:
            for cp in outp[b]:
                cp.wait()
            outp[b] = None

        def row(r, _):
            for c in range(C // L):
                sl = pl.ds(c * L, L)
                sh = pl.ds(C + c * L, L)
                fo_v[b, r, sl] = g_v[b, r, sl] + q_v[b, r, sl]
                go_v[b, r, sl] = g_v[b, r, sh] + q_v[b, r, sh]
            return 0

        lax.fori_loop(0, _F1_SUB, row, 0)
        outp[b] = (
            pltpu.async_copy(fo_v.at[b], f1o.at[pl.ds(base, _F1_SUB)], so[b]),
            pltpu.async_copy(go_v.at[b], g1o.at[pl.ds(base, _F1_SUB)], so[b]),
        )
        if s + 2 < _F1_NSUB:
            pend[b] = start(s + 2, b)
    for cps in outp:
        if cps is not None:
            for cp in cps:
                cp.wait()


_F0_SUB = 64
_F0_NSUB = N0 // NW // _F0_SUB


@functools.partial(
    pl.kernel,
    out_type=jax.ShapeDtypeStruct((N0, C), jnp.float32),
    scratch_types=[
        pltpu.VMEM((N0 // NW,), jnp.int32),
        pltpu.VMEM((2, _F0_SUB, C), jnp.float32),
        pltpu.VMEM((2, _F0_SUB, C), jnp.float32),
        pltpu.VMEM((N0 // NW // 32 + 1, 128), jnp.float32),
        pltpu.VMEM((2, 4 * C), jnp.float32),
        pltpu.SemaphoreType.DMA,
        pltpu.SemaphoreType.DMA,
        pltpu.SemaphoreType.DMA,
        pltpu.SemaphoreType.DMA,
    ],
    **_SC_MESH,
)
def _sc_feat0(g1, idx0, x04r, s0p, out, idx_v, g_v, o_v, x_v, s_v,
              sg0, sg1, so0, so1):
    w = _wid()
    sg, so = (sg0, sg1), (so0, so1)
    base0 = pl.multiple_of(w * (N0 // NW), N0 // NW)
    pltpu.sync_copy(idx0.at[pl.ds(base0, N0 // NW)], idx_v)
    pltpu.sync_copy(s0p, s_v)
    pltpu.sync_copy(
        x04r.at[pl.ds(pl.multiple_of(base0 // 32, 8), N0 // NW // 32)],
        x_v.at[pl.ds(0, N0 // NW // 32)],
    )

    def start(s, b):
        return (
            pltpu.async_copy(
                g1.at[idx_v.at[pl.ds(s * _F0_SUB, _F0_SUB)]], g_v.at[b], sg[b]
            ),
        )

    pend = [None, None]
    outp = [None, None]
    pend[0] = start(0, 0)
    pend[1] = start(1, 1)
    HALF = C // L // 2
    for s in range(_F0_NSUB):
        b = s % 2
        base = pl.multiple_of(base0 + s * _F0_SUB, _F0_SUB)
        for cp in pend[b]:
            cp.wait()
        if outp[b] is not None:
            outp[b].wait()
            outp[b] = None
        for h in range(2):
            s0 = [s_v[0, pl.ds((h * HALF + c) * L, L)] for c in range(HALF)]
            s1 = [s_v[0, pl.ds(C + (h * HALF + c) * L, L)] for c in range(HALF)]
            s2 = [s_v[0, pl.ds(2 * C + (h * HALF + c) * L, L)]
                  for c in range(HALF)]

            def row(r, _):
                g = s * _F0_SUB + r
                xrow = x_v[g // 32, pl.ds((g % 32) * 4, L)]
                x, y, z = xrow[0], xrow[1], xrow[2]
                for c in range(HALF):
                    sl = pl.ds((h * HALF + c) * L, L)
                    o_v[b, r, sl] = (g_v[b, r, sl] + x * s0[c] + y * s1[c]
                                     + z * s2[c])
                return 0

            lax.fori_loop(0, _F0_SUB, row, 0)
        outp[b] = pltpu.async_copy(o_v.at[b], out.at[pl.ds(base, _F0_SUB)],
                                   so[b])
        if s + 2 < _F0_NSUB:
            pend[b] = start(s + 2, b)
    for cp in outp:
        if cp is not None:
            cp.wait()


def kernel(xyz0, idx0, idx5, idx7, idx9, idx11, W_all, b_all, W_m4, b_m4,
           W_m3, b_m3, W_m2, b_m2, W_m1, b_m1, W_m0, b_m0, W_p4, b_p4,
           W_p3, b_p3, W_p2, b_p2, W_p1, b_p1, W_p0, b_p0):
    f32 = jnp.float32
    xyz0 = xyz0.astype(f32)
    x04r = jnp.pad(xyz0, ((0, 0), (0, 1))).reshape(N0 // 32, 128)

    i0 = idx0.astype(jnp.int32)
    i5, i7, i9, i11 = (i.astype(jnp.int32) for i in (idx5, idx7, idx9, idx11))

    x1p, x2p, x3p, x4p = _sc_gather_xyz(x04r, i11, i9, i7, i5)

    k12 = _tc_knn12(x1p, x2p)

    bias = jnp.stack([b_all, b_m4, b_m3, b_m2, b_m1, b_p4, b_p3, b_p2, b_p1,
                      b_m0, b_p0])
    feat4, feat3, feat2, T2, QR, S0 = _tc_levels(
        xyz0, x1p, x2p, x3p, x4p,
        W_all.astype(f32), W_m4.astype(f32), W_m3.astype(f32),
        W_m2.astype(f32), W_m1.astype(f32), W_m0.astype(f32),
        W_p4, W_p3, W_p2, W_p1, W_p0, bias)

    feat1, G1 = _sc_feat1g1(T2, k12, QR)

    feat0 = _sc_feat0(G1, i0, x04r, S0)

    return (feat4, feat3, feat2, feat1, feat0)
